# baseline (speedup 1.0000x reference)
import functools

import jax
import jax.numpy as jnp
from jax import lax
from jax.experimental import pallas as pl
from jax.experimental.pallas import tpu as pltpu
from jax.experimental.pallas import tpu_sc as plsc

NUM_DEV = 8
J = 2
RP = 2048
C = 1024
NC = 2
NS = 16
SROWS = 128
RPW = (RP - SROWS) // NS
RB0 = 8
RB1 = 56
RB2 = 56

_mesh = plsc.VectorSubcoreMesh(core_axis_name="c", subcore_axis_name="s")


@functools.partial(
    pl.kernel,
    mesh=_mesh,
    out_type=jax.ShapeDtypeStruct((NUM_DEV, J, RP, C), jnp.float32),
    scratch_types=[
        pltpu.VMEM_SHARED((SROWS, C), jnp.float32),
        pltpu.VMEM((16,), jnp.float32),
        pltpu.VMEM((RB2, C), jnp.float32),
        pltpu.VMEM((RB1, C), jnp.float32),
        pltpu.SemaphoreType.DMA,
        pltpu.SemaphoreType.DMA,
        pltpu.SemaphoreType.DMA,
        pltpu.SemaphoreType.DMA,
        pltpu.SemaphoreType.DMA,
        pltpu.SemaphoreType.DMA,
    ],
)
def _sc_select_broadcast(src, msk, out, spbuf, mbuf, buf0, buf1,
                         gsem0, gsem1, wsem0, wsem1, spg, spw):
    cid = lax.axis_index("c")
    sid = lax.axis_index("s")
    j = cid
    rbase = sid * RPW

    pltpu.sync_copy(msk, mbuf.at[pl.ds(0, NUM_DEV)])
    m = mbuf[...]
    best = m[0]
    bi = jnp.int32(0)
    for i in range(1, NUM_DEV):
        v = m[i]
        p = v > best
        bi = lax.select(p, jnp.int32(i), bi)
        best = lax.select(p, v, best)

    @pl.when(sid == 0)
    def _spmem_fanout():
        pltpu.async_copy(
            src.at[bi, j, pl.ds(RP - SROWS, SROWS)], spbuf, spg).wait()
        for d in range(NUM_DEV):
            pltpu.async_copy(
                spbuf, out.at[d, j, pl.ds(RP - SROWS, SROWS)], spw)

    r0, r1, r2 = rbase, rbase + RB0, rbase + RB0 + RB1
    bramp = buf0.at[pl.ds(0, RB0)]

    g0 = pltpu.async_copy(src.at[bi, j, pl.ds(r0, RB0)], bramp, gsem0)
    g1 = pltpu.async_copy(src.at[bi, j, pl.ds(r1, RB1)], buf1, gsem1)
    g0.wait()
    w0 = [pltpu.async_copy(bramp, out.at[d, j, pl.ds(r0, RB0)], wsem0)
          for d in range(NUM_DEV)]
    g1.wait()
    w1 = [pltpu.async_copy(buf1, out.at[d, j, pl.ds(r1, RB1)], wsem1)
          for d in range(NUM_DEV)]
    for h in w0:
        h.wait()
    g2 = pltpu.async_copy(src.at[bi, j, pl.ds(r2, RB2)], buf0, gsem0)
    for h in w1:
        h.wait()
    g2.wait()
    w2 = [pltpu.async_copy(buf0, out.at[d, j, pl.ds(r2, RB2)], wsem0)
          for d in range(NUM_DEV)]
    for h in w2:
        h.wait()

    @pl.when(sid == 0)
    def _spmem_drain():
        for d in range(NUM_DEV):
            pltpu.make_async_copy(
                spbuf, out.at[d, j, pl.ds(RP - SROWS, SROWS)], spw).wait()


def kernel(mesh_tensor, mask):
    return _sc_select_broadcast(mesh_tensor, mask)


# speedup vs baseline: 1.0243x; 1.0243x over previous
import functools

import jax
import jax.numpy as jnp
from jax import lax
from jax.experimental import pallas as pl
from jax.experimental.pallas import tpu as pltpu
from jax.experimental.pallas import tpu_sc as plsc

NUM_DEV = 8
J = 2
RP = 2048
C = 1024
NC = 2
NS = 16
NW = NC * NS
RPW = RP * J // NW
RB0 = 8
RB1 = 64
RB2 = 56

_mesh = plsc.VectorSubcoreMesh(core_axis_name="c", subcore_axis_name="s")


@functools.partial(
    pl.kernel,
    mesh=_mesh,
    out_type=jax.ShapeDtypeStruct((NUM_DEV, J, RP, C), jnp.float32),
    scratch_types=[
        pltpu.VMEM((16,), jnp.float32),
        pltpu.VMEM((RB2, C), jnp.float32),
        pltpu.VMEM((RB1, C), jnp.float32),
        pltpu.SemaphoreType.DMA,
        pltpu.SemaphoreType.DMA,
        pltpu.SemaphoreType.DMA,
        pltpu.SemaphoreType.DMA,
    ],
)
def _sc_select_broadcast(src, msk, out, mbuf, buf0, buf1,
                         gsem0, gsem1, wsem0, wsem1):
    wid = lax.axis_index("s") * NC + lax.axis_index("c")
    j = wid % J
    rbase = (wid // J) * RPW

    pltpu.sync_copy(msk, mbuf.at[pl.ds(0, NUM_DEV)])
    m = mbuf[...]
    best = m[0]
    bi = jnp.int32(0)
    for i in range(1, NUM_DEV):
        v = m[i]
        p = v > best
        bi = lax.select(p, jnp.int32(i), bi)
        best = lax.select(p, v, best)

    r0, r1, r2 = rbase, rbase + RB0, rbase + RB0 + RB1
    bramp = buf0.at[pl.ds(0, RB0)]

    g0 = pltpu.async_copy(src.at[bi, j, pl.ds(r0, RB0)], bramp, gsem0)
    g1 = pltpu.async_copy(src.at[bi, j, pl.ds(r1, RB1)], buf1, gsem1)
    g0.wait()
    w0 = [pltpu.async_copy(bramp, out.at[d, j, pl.ds(r0, RB0)], wsem0)
          for d in range(NUM_DEV)]
    g1.wait()
    w1 = [pltpu.async_copy(buf1, out.at[d, j, pl.ds(r1, RB1)], wsem1)
          for d in range(NUM_DEV)]
    for h in w0:
        h.wait()
    g2 = pltpu.async_copy(src.at[bi, j, pl.ds(r2, RB2)], buf0, gsem0)
    for h in w1:
        h.wait()
    g2.wait()
    w2 = [pltpu.async_copy(buf0, out.at[d, j, pl.ds(r2, RB2)], wsem0)
          for d in range(NUM_DEV)]
    for h in w2:
        h.wait()


def kernel(mesh_tensor, mask):
    return _sc_select_broadcast(mesh_tensor, mask)
